# baseline (speedup 1.0000x reference)
import functools

import jax
import jax.numpy as jnp
from jax import lax
from jax.experimental import pallas as pl
from jax.experimental.pallas import tpu as pltpu
from jax.experimental.pallas import tpu_sc as plsc

N_LAYERS = 3
D_NODE = 128
D_EDGE = 64
CUTOFF = 10.0

_EB = 1024
_NB = 512


def _ln(x, g, b):
    mu = jnp.mean(x, axis=-1, keepdims=True)
    v = jnp.mean((x - mu) ** 2, axis=-1, keepdims=True)
    return (x - mu) / jnp.sqrt(v + 1e-5) * g + b


def _silu(x):
    return x * (1.0 / (1.0 + jnp.exp(-x)))


def _edge_mlp_body(el_ref, off_ref, w0_ref, b0_ref, g0_ref, bt0_ref,
                   w1_ref, b1_ref, out_ref):
    el = jnp.sqrt(el_ref[...])
    off = off_ref[...]
    step = CUTOFF / (D_EDGE - 1)
    coeff = -0.5 / (step * step)
    d = el - off
    ea = jnp.exp(coeff * d * d)
    e = jnp.dot(ea, w0_ref[...], preferred_element_type=jnp.float32) + b0_ref[...]
    e = _silu(_ln(e, g0_ref[...], bt0_ref[...]))
    e = jnp.dot(e, w1_ref[...], preferred_element_type=jnp.float32) + b1_ref[...]
    out_ref[...] = e


def _edge_mlp_layer(el2, off, c):
    e_pad = el2.shape[0]
    nblk = e_pad // _EB
    vspec = pl.BlockSpec((1, D_NODE), lambda b: (0, 0))
    return pl.pallas_call(
        _edge_mlp_body,
        grid=(nblk,),
        in_specs=[
            pl.BlockSpec((_EB, 1), lambda b: (b, 0)),
            pl.BlockSpec((1, D_EDGE), lambda b: (0, 0)),
            pl.BlockSpec((D_EDGE, D_NODE), lambda b: (0, 0)),
            vspec, vspec, vspec,
            pl.BlockSpec((D_NODE, D_NODE), lambda b: (0, 0)),
            vspec,
        ],
        out_specs=pl.BlockSpec((_EB, D_NODE), lambda b: (b, 0)),
        out_shape=jax.ShapeDtypeStruct((e_pad, D_NODE), jnp.float32),
    )(el2, off, c['eW0'], c['eb0'].reshape(1, D_NODE),
      c['eg0'].reshape(1, D_NODE), c['ebt0'].reshape(1, D_NODE), c['eW1'],
      c['eb1'].reshape(1, D_NODE))


_SC_C = 64
_ZB = 16
_LC = 128


def _sc_edge_len2(px, py, pz, src_full, dst_full):
    e_padf = src_full.shape[0]
    nch = e_padf // (32 * _LC)

    def body(px_h, py_h, pz_h, src_h, dst_h, out_h,
             sidx, didx, sx, sy, sz, dx, dy, dz, l2b, sem):
        c = lax.axis_index("c")
        s = lax.axis_index("s")
        w = s * 2 + c

        def chunk(t, carry):
            base = (w * nch + t) * _LC
            pltpu.sync_copy(src_h.at[pl.ds(base, _LC)], sidx)
            pltpu.sync_copy(dst_h.at[pl.ds(base, _LC)], didx)
            cps = [pltpu.async_copy(px_h.at[sidx], sx, sem),
                   pltpu.async_copy(py_h.at[sidx], sy, sem),
                   pltpu.async_copy(pz_h.at[sidx], sz, sem),
                   pltpu.async_copy(px_h.at[didx], dx, sem),
                   pltpu.async_copy(py_h.at[didx], dy, sem),
                   pltpu.async_copy(pz_h.at[didx], dz, sem)]
            for cp in cps:
                cp.wait()

            def cb(j, inner):
                sl = pl.ds(j * 16, 16)
                a = sx[sl] - dx[sl]
                b = sy[sl] - dy[sl]
                g = sz[sl] - dz[sl]
                l2b[sl] = a * a + b * b + g * g
                return inner

            lax.fori_loop(0, _LC // 16, cb, 0)
            pltpu.sync_copy(l2b, out_h.at[pl.ds(base, _LC)])
            return carry

        lax.fori_loop(0, nch, chunk, 0)

    mesh = plsc.VectorSubcoreMesh(core_axis_name="c", subcore_axis_name="s")
    f = pl.kernel(
        body,
        out_type=jax.ShapeDtypeStruct((e_padf,), jnp.float32),
        mesh=mesh,
        scratch_types=[
            pltpu.VMEM((_LC,), jnp.int32),
            pltpu.VMEM((_LC,), jnp.int32),
            pltpu.VMEM((_LC,), jnp.float32),
            pltpu.VMEM((_LC,), jnp.float32),
            pltpu.VMEM((_LC,), jnp.float32),
            pltpu.VMEM((_LC,), jnp.float32),
            pltpu.VMEM((_LC,), jnp.float32),
            pltpu.VMEM((_LC,), jnp.float32),
            pltpu.VMEM((_LC,), jnp.float32),
            pltpu.SemaphoreType.DMA,
        ],
    )
    return f(px, py, pz, src_full, dst_full)


def _sc_scatter(x_pad, e_l, src_pad, dst_pad, n_rows, nchunks):
    nc, ns = 2, 16
    zr = n_rows // ns

    def body(x_hbm, e_hbm, src_hbm, dst_hbm, out_hbm,
             idx0, idx1, idx2, dstv0, dstv1, dstv2, xb0, xb1, xb2,
             eb0, eb1, zbuf, acc_sh,
             semf0, semf1, semf2, seme0, seme1, semg0, semg1, semg2, sems):
        idx = [idx0, idx1, idx2]
        dstv = [dstv0, dstv1, dstv2]
        xb = [xb0, xb1, xb2]
        eb = [eb0, eb1]
        semf = [semf0, semf1, semf2]
        seme = [seme0, seme1]
        semg = [semg0, semg1, semg2]
        c = lax.axis_index("c")
        s = lax.axis_index("s")

        def zero_body(j, carry):
            for k in range(D_NODE // 16):
                sl = pl.ds(k * 16, 16)
                zbuf[j, sl] = jnp.zeros((16,), jnp.float32)
            return carry

        lax.fori_loop(0, _ZB, zero_body, 0)
        for k in range(zr // _ZB):
            pltpu.sync_copy(zbuf, acc_sh.at[pl.ds(s * zr + k * _ZB, _ZB)])
        plsc.subcore_barrier()

        w = s * nc + c

        def ebase(t):
            return (w * nchunks + t) * _SC_C

        def issue_fetch_idx(t, a):
            pltpu.async_copy(src_hbm.at[pl.ds(ebase(t), _SC_C)], idx[a], semf[a])
            pltpu.async_copy(dst_hbm.at[pl.ds(ebase(t), _SC_C)], dstv[a], semf[a])

        def wait_fetch_idx(t, a):
            pltpu.make_async_copy(src_hbm.at[pl.ds(ebase(t), _SC_C)], idx[a], semf[a]).wait()
            pltpu.make_async_copy(dst_hbm.at[pl.ds(ebase(t), _SC_C)], dstv[a], semf[a]).wait()

        def issue_fetch_e(t, b):
            pltpu.async_copy(e_hbm.at[pl.ds(ebase(t), _SC_C)], eb[b], seme[b])

        def wait_fetch_e(t, b):
            pltpu.make_async_copy(e_hbm.at[pl.ds(ebase(t), _SC_C)], eb[b], seme[b]).wait()

        def issue_gather(a):
            pltpu.async_copy(x_hbm.at[idx[a]], xb[a], semg[a])

        def wait_gather(a):
            pltpu.make_async_copy(x_hbm.at[idx[a]], xb[a], semg[a]).wait()

        def issue_scatter(a):
            pltpu.async_copy(xb[a], acc_sh.at[dstv[a]], sems, add=True)

        def wait_scatter(a):
            pltpu.make_async_copy(xb[a], acc_sh.at[dstv[a]], sems).wait()

        issue_fetch_idx(0, 0)
        issue_fetch_idx(1, 1)
        issue_fetch_e(0, 0)
        wait_fetch_idx(0, 0)
        issue_gather(0)
        wait_fetch_idx(1, 1)
        issue_gather(1)

        @pl.loop(0, nchunks, step=6)
        def _(t0):
            for dt in range(6):
                t = t0 + dt
                a = dt % 3
                an2 = (dt + 2) % 3
                b = dt % 2
                bn = (dt + 1) % 2

                @pl.when(t > 0)
                def _():
                    wait_scatter((dt + 2) % 3)

                @pl.when(t + 2 < nchunks)
                def _():
                    issue_fetch_idx(t + 2, an2)

                @pl.when(t + 1 < nchunks)
                def _():
                    issue_fetch_e(t + 1, bn)

                wait_gather(a)
                wait_fetch_e(t, b)

                def mul_body(j, inner):
                    for k in range(D_NODE // 16):
                        sl = pl.ds(k * 16, 16)
                        xb[a][j, sl] = xb[a][j, sl] * eb[b][j, sl]
                    return inner

                lax.fori_loop(0, _SC_C, mul_body, 0)
                issue_scatter(a)

                @pl.when(t + 2 < nchunks)
                def _():
                    wait_fetch_idx(t + 2, an2)
                    issue_gather(an2)

        wait_scatter((nchunks - 1) % 3)
        plsc.subcore_barrier()
        pltpu.sync_copy(acc_sh.at[pl.ds(s * zr, zr)],
                        out_hbm.at[c, pl.ds(s * zr, zr)])

    mesh = plsc.VectorSubcoreMesh(core_axis_name="c", subcore_axis_name="s")
    f = pl.kernel(
        body,
        out_type=jax.ShapeDtypeStruct((nc, n_rows, D_NODE), jnp.float32),
        mesh=mesh,
        scratch_types=[
            pltpu.VMEM((_SC_C,), jnp.int32),
            pltpu.VMEM((_SC_C,), jnp.int32),
            pltpu.VMEM((_SC_C,), jnp.int32),
            pltpu.VMEM((_SC_C,), jnp.int32),
            pltpu.VMEM((_SC_C,), jnp.int32),
            pltpu.VMEM((_SC_C,), jnp.int32),
            pltpu.VMEM((_SC_C, D_NODE), jnp.float32),
            pltpu.VMEM((_SC_C, D_NODE), jnp.float32),
            pltpu.VMEM((_SC_C, D_NODE), jnp.float32),
            pltpu.VMEM((_SC_C, D_NODE), jnp.float32),
            pltpu.VMEM((_SC_C, D_NODE), jnp.float32),
            pltpu.VMEM((_ZB, D_NODE), jnp.float32),
            pltpu.VMEM_SHARED((n_rows, D_NODE), jnp.float32),
            pltpu.SemaphoreType.DMA,
            pltpu.SemaphoreType.DMA,
            pltpu.SemaphoreType.DMA,
            pltpu.SemaphoreType.DMA,
            pltpu.SemaphoreType.DMA,
            pltpu.SemaphoreType.DMA,
            pltpu.SemaphoreType.DMA,
            pltpu.SemaphoreType.DMA,
            pltpu.SemaphoreType.DMA,
        ],
    )
    return f(x_pad, e_l, src_pad, dst_pad)


def _node_mlp_body(is_last, n_valid, x_ref, h0a_ref, h0b_ref, es_ref, ca_ref,
                   w0_ref, b0_ref, g0_ref, bt0_ref, w1_ref, b1_ref,
                   g1_ref, bt1_ref, hn_ref, s1_ref, s2_ref):
    x = x_ref[...]
    h0 = h0a_ref[0] + h0b_ref[0] + x * es_ref[...]
    h = ca_ref[...] * h0 + x
    h = jnp.dot(h, w0_ref[...], preferred_element_type=jnp.float32) + b0_ref[...]
    h = _silu(_ln(h, g0_ref[...], bt0_ref[...]))
    h = jnp.dot(h, w1_ref[...], preferred_element_type=jnp.float32) + b1_ref[...]
    if not is_last:
        h = _silu(_ln(h, g1_ref[...], bt1_ref[...]))
    hn_ref[...] = h
    row = jax.lax.broadcasted_iota(jnp.int32, h.shape, 0) + pl.program_id(0) * _NB
    hm = jnp.where(row < n_valid, h, 0.0)
    s1_ref[...] = jnp.sum(hm, axis=0, keepdims=True)[None]
    s2_ref[...] = jnp.sum(hm * hm, axis=0, keepdims=True)[None]


def _node_mlp(x_pad, h0_pair, es_pad, c, is_last, n_valid):
    n_pad = x_pad.shape[0]
    nblk = n_pad // _NB
    vspec = pl.BlockSpec((1, D_NODE), lambda b: (0, 0))
    hn, s1, s2 = pl.pallas_call(
        functools.partial(_node_mlp_body, is_last, n_valid),
        grid=(nblk,),
        in_specs=[
            pl.BlockSpec((_NB, D_NODE), lambda b: (b, 0)),
            pl.BlockSpec((1, _NB, D_NODE), lambda b: (0, b, 0)),
            pl.BlockSpec((1, _NB, D_NODE), lambda b: (1, b, 0)),
            pl.BlockSpec((_NB, D_NODE), lambda b: (b, 0)),
            vspec,
            pl.BlockSpec((D_NODE, D_NODE), lambda b: (0, 0)),
            vspec, vspec, vspec,
            pl.BlockSpec((D_NODE, D_NODE), lambda b: (0, 0)),
            vspec, vspec, vspec,
        ],
        out_specs=[
            pl.BlockSpec((_NB, D_NODE), lambda b: (b, 0)),
            pl.BlockSpec((1, 1, D_NODE), lambda b: (b, 0, 0)),
            pl.BlockSpec((1, 1, D_NODE), lambda b: (b, 0, 0)),
        ],
        out_shape=[
            jax.ShapeDtypeStruct((n_pad, D_NODE), jnp.float32),
            jax.ShapeDtypeStruct((nblk, 1, D_NODE), jnp.float32),
            jax.ShapeDtypeStruct((nblk, 1, D_NODE), jnp.float32),
        ],
    )(x_pad, h0_pair, h0_pair, es_pad,
      c['calpha'].reshape(1, D_NODE), c['nW0'], c['nb0'].reshape(1, D_NODE),
      c['ng0'].reshape(1, D_NODE), c['nbt0'].reshape(1, D_NODE), c['nW1'],
      c['nb1'].reshape(1, D_NODE), c['ng1'].reshape(1, D_NODE),
      c['nbt1'].reshape(1, D_NODE))
    return hn, s1, s2


def _norm_body(hn_ref, x_ref, a_ref, bb_ref, al_ref, out_ref):
    out_ref[...] = hn_ref[...] * a_ref[...] + x_ref[...] * al_ref[...] + bb_ref[...]


def _norm_residual(hn_pad, x_pad, a, bb, alpha):
    n_pad = x_pad.shape[0]
    nblk = n_pad // _NB
    vspec = pl.BlockSpec((1, D_NODE), lambda b: (0, 0))
    alv = jnp.broadcast_to(alpha.reshape(1, 1), (1, D_NODE))
    return pl.pallas_call(
        _norm_body,
        grid=(nblk,),
        in_specs=[
            pl.BlockSpec((_NB, D_NODE), lambda b: (b, 0)),
            pl.BlockSpec((_NB, D_NODE), lambda b: (b, 0)),
            vspec, vspec, vspec,
        ],
        out_specs=pl.BlockSpec((_NB, D_NODE), lambda b: (b, 0)),
        out_shape=jax.ShapeDtypeStruct((n_pad, D_NODE), jnp.float32),
    )(hn_pad, x_pad, a.reshape(1, D_NODE), bb.reshape(1, D_NODE), alv)


def _pad_to(x, m, axis=0, value=0):
    n = x.shape[axis]
    p = (-n) % m
    if p == 0:
        return x
    widths = [(0, 0)] * x.ndim
    widths[axis] = (0, p)
    return jnp.pad(x, widths, constant_values=value)


def kernel(atom_type, pos, batch, edge_index, params):
    n = atom_type.shape[0]
    e_total = edge_index.shape[1]
    e_neigh = e_total - n
    src = edge_index[0]
    dst = edge_index[1]

    e_padf = -(-e_total // (32 * _LC)) * (32 * _LC)
    src_full = _pad_to(src, e_padf, value=0)[:e_padf]
    dst_full = _pad_to(dst, e_padf, value=0)[:e_padf]
    l2 = _sc_edge_len2(pos[:, 0], pos[:, 1], pos[:, 2], src_full, dst_full)
    el2 = l2.reshape(e_padf, 1)
    off = (jnp.arange(D_EDGE, dtype=jnp.float32) * (CUTOFF / (D_EDGE - 1)))
    off = off.reshape(1, D_EDGE)
    e_layers = [_edge_mlp_layer(el2, off, params['convs'][i])
                for i in range(N_LAYERS)]

    x = params['emb'][atom_type]
    x_pad = _pad_to(x, _NB)
    n_pad = x_pad.shape[0]

    per = 32 * _SC_C
    nchunks = -(-(-(-e_neigh // per)) // 6) * 6
    e32 = nchunks * per
    dump_row = n_pad - 8
    src_pad = _pad_to(src[:e_neigh], e32, value=0)[:e32]
    dst_pad = _pad_to(dst[:e_neigh], e32, value=dump_row)[:e32]

    for i in range(N_LAYERS):
        c = params['convs'][i]
        h0_pair = _sc_scatter(x_pad, e_layers[i], src_pad, dst_pad, n_pad,
                              nchunks)
        es_pad = _pad_to(e_layers[i][e_neigh:e_total], _NB)

        hn_pad, s1, s2 = _node_mlp(x_pad, h0_pair, es_pad, c,
                                   i + 1 == N_LAYERS, n)
        gn = params['gn'][i]
        mu = jnp.sum(s1, axis=(0, 1)) / n
        m2 = jnp.sum(s2, axis=(0, 1)) / n
        ms = gn['ms']
        var = m2 - 2.0 * ms * mu * mu + ms * ms * mu * mu
        inv = 1.0 / jnp.sqrt(var + 1e-5)
        a = inv * gn['g']
        bb = gn['b'] - ms * mu * inv * gn['g']
        x_pad = _norm_residual(hn_pad, x_pad, a, bb, params['alpha'][i])

    return x_pad[:n]


# speedup vs baseline: 1.1553x; 1.1553x over previous
import functools

import jax
import jax.numpy as jnp
from jax import lax
from jax.experimental import pallas as pl
from jax.experimental.pallas import tpu as pltpu
from jax.experimental.pallas import tpu_sc as plsc

N_LAYERS = 3
D_NODE = 128
D_EDGE = 64
CUTOFF = 10.0

_EB = 1024
_NB = 512


def _ln(x, g, b):
    mu = jnp.mean(x, axis=-1, keepdims=True)
    v = jnp.mean((x - mu) ** 2, axis=-1, keepdims=True)
    return (x - mu) / jnp.sqrt(v + 1e-5) * g + b


def _silu(x):
    return x * (1.0 / (1.0 + jnp.exp(-x)))


def _edge_mlp_body(el_ref, off_ref, w0_ref, b0_ref, g0_ref, bt0_ref,
                   w1_ref, b1_ref, out_ref):
    el = jnp.sqrt(el_ref[...])
    off = off_ref[...]
    step = CUTOFF / (D_EDGE - 1)
    coeff = -0.5 / (step * step)
    d = el - off
    ea = jnp.exp(coeff * d * d)
    e = jnp.dot(ea, w0_ref[...], preferred_element_type=jnp.float32) + b0_ref[...]
    e = _silu(_ln(e, g0_ref[...], bt0_ref[...]))
    e = jnp.dot(e, w1_ref[...], preferred_element_type=jnp.float32) + b1_ref[...]
    out_ref[...] = e


def _edge_mlp_layer(el2, off, c):
    e_pad = el2.shape[0]
    nblk = e_pad // _EB
    vspec = pl.BlockSpec((1, D_NODE), lambda b: (0, 0))
    return pl.pallas_call(
        _edge_mlp_body,
        grid=(nblk,),
        in_specs=[
            pl.BlockSpec((_EB, 1), lambda b: (b, 0)),
            pl.BlockSpec((1, D_EDGE), lambda b: (0, 0)),
            pl.BlockSpec((D_EDGE, D_NODE), lambda b: (0, 0)),
            vspec, vspec, vspec,
            pl.BlockSpec((D_NODE, D_NODE), lambda b: (0, 0)),
            vspec,
        ],
        out_specs=pl.BlockSpec((_EB, D_NODE), lambda b: (b, 0)),
        out_shape=jax.ShapeDtypeStruct((e_pad, D_NODE), jnp.float32),
    )(el2, off, c['eW0'], c['eb0'].reshape(1, D_NODE),
      c['eg0'].reshape(1, D_NODE), c['ebt0'].reshape(1, D_NODE), c['eW1'],
      c['eb1'].reshape(1, D_NODE))


_SC_C = 64
_ZB = 16
_LC = 128


def _sc_edge_len2(px, py, pz, src_full, dst_full):
    e_padf = src_full.shape[0]
    nch = e_padf // (32 * _LC)

    def body(px_h, py_h, pz_h, src_h, dst_h, out_h,
             sidx, didx, sx, sy, sz, dx, dy, dz, l2b, sem):
        c = lax.axis_index("c")
        s = lax.axis_index("s")
        w = s * 2 + c

        def chunk(t, carry):
            base = (w * nch + t) * _LC
            pltpu.sync_copy(src_h.at[pl.ds(base, _LC)], sidx)
            pltpu.sync_copy(dst_h.at[pl.ds(base, _LC)], didx)
            cps = [pltpu.async_copy(px_h.at[sidx], sx, sem),
                   pltpu.async_copy(py_h.at[sidx], sy, sem),
                   pltpu.async_copy(pz_h.at[sidx], sz, sem),
                   pltpu.async_copy(px_h.at[didx], dx, sem),
                   pltpu.async_copy(py_h.at[didx], dy, sem),
                   pltpu.async_copy(pz_h.at[didx], dz, sem)]
            for cp in cps:
                cp.wait()

            def cb(j, inner):
                sl = pl.ds(j * 16, 16)
                a = sx[sl] - dx[sl]
                b = sy[sl] - dy[sl]
                g = sz[sl] - dz[sl]
                l2b[sl] = a * a + b * b + g * g
                return inner

            lax.fori_loop(0, _LC // 16, cb, 0)
            pltpu.sync_copy(l2b, out_h.at[pl.ds(base, _LC)])
            return carry

        lax.fori_loop(0, nch, chunk, 0)

    mesh = plsc.VectorSubcoreMesh(core_axis_name="c", subcore_axis_name="s")
    f = pl.kernel(
        body,
        out_type=jax.ShapeDtypeStruct((e_padf,), jnp.float32),
        mesh=mesh,
        scratch_types=[
            pltpu.VMEM((_LC,), jnp.int32),
            pltpu.VMEM((_LC,), jnp.int32),
            pltpu.VMEM((_LC,), jnp.float32),
            pltpu.VMEM((_LC,), jnp.float32),
            pltpu.VMEM((_LC,), jnp.float32),
            pltpu.VMEM((_LC,), jnp.float32),
            pltpu.VMEM((_LC,), jnp.float32),
            pltpu.VMEM((_LC,), jnp.float32),
            pltpu.VMEM((_LC,), jnp.float32),
            pltpu.SemaphoreType.DMA,
        ],
    )
    return f(px, py, pz, src_full, dst_full)


def _sc_scatter(x_pad, e_l, src_pad, dst_pad, n_rows, acc_rows, nchunks):
    nc, ns = 2, 16
    zr = acc_rows // ns

    def body(x_hbm, e_hbm, src_hbm, dst_hbm, out_hbm,
             idx0, idx1, idx2, idx3, dstv0, dstv1, dstv2, dstv3, xb0, xb1,
             eb0, eb1, mb0, mb1, acc_sh,
             semf0, semf1, semf2, semf3, semd0, semd1, semd2, semd3,
             seme0, seme1, semg0, semg1, sems0, sems1):
        idx = [idx0, idx1, idx2, idx3]
        dstv = [dstv0, dstv1, dstv2, dstv3]
        xb = [xb0, xb1]
        eb = [eb0, eb1]
        mb = [mb0, mb1]
        semf = [semf0, semf1, semf2, semf3]
        semd = [semd0, semd1, semd2, semd3]
        seme = [seme0, seme1]
        semg = [semg0, semg1]
        sems = [sems0, sems1]
        c = lax.axis_index("c")
        s = lax.axis_index("s")

        def zero_body(j, carry):
            for k in range(D_NODE // 16):
                sl = pl.ds(k * 16, 16)
                mb0[j, sl] = jnp.zeros((16,), jnp.float32)
            return carry

        lax.fori_loop(0, _SC_C, zero_body, 0)
        nz = zr // _SC_C
        for k in range(nz):
            pltpu.sync_copy(mb0, acc_sh.at[pl.ds(s * zr + k * _SC_C, _SC_C)])
        rem = zr - nz * _SC_C
        if rem:
            pltpu.sync_copy(mb0.at[pl.ds(0, rem)],
                            acc_sh.at[pl.ds(s * zr + nz * _SC_C, rem)])
        plsc.subcore_barrier()

        w = s * nc + c

        def ebase(t):
            return (w * nchunks + t) * _SC_C

        def issue_fetch_idx(t, a):
            pltpu.async_copy(src_hbm.at[pl.ds(ebase(t), _SC_C)], idx[a], semf[a])
            pltpu.async_copy(dst_hbm.at[pl.ds(ebase(t), _SC_C)], dstv[a], semd[a])

        def wait_fetch_idx(t, a):
            pltpu.make_async_copy(src_hbm.at[pl.ds(ebase(t), _SC_C)], idx[a], semf[a]).wait()
            pltpu.make_async_copy(dst_hbm.at[pl.ds(ebase(t), _SC_C)], dstv[a], semd[a]).wait()

        def issue_fetch_e(t, b):
            pltpu.async_copy(e_hbm.at[pl.ds(ebase(t), _SC_C)], eb[b], seme[b])

        def wait_fetch_e(t, b):
            pltpu.make_async_copy(e_hbm.at[pl.ds(ebase(t), _SC_C)], eb[b], seme[b]).wait()

        def issue_gather(a, b):
            pltpu.async_copy(x_hbm.at[idx[a]], xb[b], semg[b])

        def wait_gather(a, b):
            pltpu.make_async_copy(x_hbm.at[idx[a]], xb[b], semg[b]).wait()

        def issue_scatter(b, d):
            pltpu.async_copy(mb[b], acc_sh.at[dstv[d]], sems[b], add=True)

        def wait_scatter(b, d):
            pltpu.make_async_copy(mb[b], acc_sh.at[dstv[d]], sems[b]).wait()

        issue_fetch_idx(0, 0)
        issue_fetch_idx(1, 1)
        issue_fetch_e(0, 0)
        issue_fetch_e(1, 1)
        wait_fetch_idx(0, 0)
        issue_gather(0, 0)
        wait_fetch_idx(1, 1)
        issue_gather(1, 1)

        @pl.loop(0, nchunks, step=4)
        def _(t0):
            for dt in range(4):
                t = t0 + dt
                b = dt % 2
                d = dt
                dn2 = (dt + 2) % 4

                @pl.when(t >= 2)
                def _():
                    wait_scatter(b, dn2)

                @pl.when(t + 2 < nchunks)
                def _():
                    issue_fetch_idx(t + 2, dn2)

                wait_gather(d, b)
                wait_fetch_e(t, b)

                def mul_body(j, inner):
                    for k in range(D_NODE // 16):
                        sl = pl.ds(k * 16, 16)
                        mb[b][j, sl] = xb[b][j, sl] * eb[b][j, sl]
                    return inner

                lax.fori_loop(0, _SC_C, mul_body, 0)
                issue_scatter(b, d)

                @pl.when(t + 2 < nchunks)
                def _():
                    issue_fetch_e(t + 2, b)
                    wait_fetch_idx(t + 2, dn2)
                    issue_gather(dn2, b)

        wait_scatter((nchunks - 2) % 2, (nchunks - 2) % 4)
        wait_scatter((nchunks - 1) % 2, (nchunks - 1) % 4)
        plsc.subcore_barrier()
        pltpu.sync_copy(acc_sh.at[pl.ds(s * zr, zr)],
                        out_hbm.at[c, pl.ds(s * zr, zr)])

    mesh = plsc.VectorSubcoreMesh(core_axis_name="c", subcore_axis_name="s")
    f = pl.kernel(
        body,
        out_type=jax.ShapeDtypeStruct((nc, n_rows, D_NODE), jnp.float32),
        mesh=mesh,
        scratch_types=(
            [pltpu.VMEM((_SC_C,), jnp.int32) for _ in range(8)]
            + [pltpu.VMEM((_SC_C, D_NODE), jnp.float32) for _ in range(6)]
            + [pltpu.VMEM_SHARED((acc_rows, D_NODE), jnp.float32)]
            + [pltpu.SemaphoreType.DMA for _ in range(14)]
        ),
    )
    return f(x_pad, e_l, src_pad, dst_pad)


def _node_mlp_body(is_last, n_valid, x_ref, h0a_ref, h0b_ref, es_ref, ca_ref,
                   w0_ref, b0_ref, g0_ref, bt0_ref, w1_ref, b1_ref,
                   g1_ref, bt1_ref, hn_ref, s1_ref, s2_ref):
    x = x_ref[...]
    h0 = h0a_ref[0] + h0b_ref[0] + x * es_ref[...]
    h = ca_ref[...] * h0 + x
    h = jnp.dot(h, w0_ref[...], preferred_element_type=jnp.float32) + b0_ref[...]
    h = _silu(_ln(h, g0_ref[...], bt0_ref[...]))
    h = jnp.dot(h, w1_ref[...], preferred_element_type=jnp.float32) + b1_ref[...]
    if not is_last:
        h = _silu(_ln(h, g1_ref[...], bt1_ref[...]))
    hn_ref[...] = h
    row = jax.lax.broadcasted_iota(jnp.int32, h.shape, 0) + pl.program_id(0) * _NB
    hm = jnp.where(row < n_valid, h, 0.0)
    s1_ref[...] = jnp.sum(hm, axis=0, keepdims=True)[None]
    s2_ref[...] = jnp.sum(hm * hm, axis=0, keepdims=True)[None]


def _node_mlp(x_pad, h0_pair, es_pad, c, is_last, n_valid):
    n_pad = x_pad.shape[0]
    nblk = n_pad // _NB
    vspec = pl.BlockSpec((1, D_NODE), lambda b: (0, 0))
    hn, s1, s2 = pl.pallas_call(
        functools.partial(_node_mlp_body, is_last, n_valid),
        grid=(nblk,),
        in_specs=[
            pl.BlockSpec((_NB, D_NODE), lambda b: (b, 0)),
            pl.BlockSpec((1, _NB, D_NODE), lambda b: (0, b, 0)),
            pl.BlockSpec((1, _NB, D_NODE), lambda b: (1, b, 0)),
            pl.BlockSpec((_NB, D_NODE), lambda b: (b, 0)),
            vspec,
            pl.BlockSpec((D_NODE, D_NODE), lambda b: (0, 0)),
            vspec, vspec, vspec,
            pl.BlockSpec((D_NODE, D_NODE), lambda b: (0, 0)),
            vspec, vspec, vspec,
        ],
        out_specs=[
            pl.BlockSpec((_NB, D_NODE), lambda b: (b, 0)),
            pl.BlockSpec((1, 1, D_NODE), lambda b: (b, 0, 0)),
            pl.BlockSpec((1, 1, D_NODE), lambda b: (b, 0, 0)),
        ],
        out_shape=[
            jax.ShapeDtypeStruct((n_pad, D_NODE), jnp.float32),
            jax.ShapeDtypeStruct((nblk, 1, D_NODE), jnp.float32),
            jax.ShapeDtypeStruct((nblk, 1, D_NODE), jnp.float32),
        ],
    )(x_pad, h0_pair, h0_pair, es_pad,
      c['calpha'].reshape(1, D_NODE), c['nW0'], c['nb0'].reshape(1, D_NODE),
      c['ng0'].reshape(1, D_NODE), c['nbt0'].reshape(1, D_NODE), c['nW1'],
      c['nb1'].reshape(1, D_NODE), c['ng1'].reshape(1, D_NODE),
      c['nbt1'].reshape(1, D_NODE))
    return hn, s1, s2


def _norm_body(hn_ref, x_ref, a_ref, bb_ref, al_ref, out_ref):
    out_ref[...] = hn_ref[...] * a_ref[...] + x_ref[...] * al_ref[...] + bb_ref[...]


def _norm_residual(hn_pad, x_pad, a, bb, alpha):
    n_pad = x_pad.shape[0]
    nblk = n_pad // _NB
    vspec = pl.BlockSpec((1, D_NODE), lambda b: (0, 0))
    alv = jnp.broadcast_to(alpha.reshape(1, 1), (1, D_NODE))
    return pl.pallas_call(
        _norm_body,
        grid=(nblk,),
        in_specs=[
            pl.BlockSpec((_NB, D_NODE), lambda b: (b, 0)),
            pl.BlockSpec((_NB, D_NODE), lambda b: (b, 0)),
            vspec, vspec, vspec,
        ],
        out_specs=pl.BlockSpec((_NB, D_NODE), lambda b: (b, 0)),
        out_shape=jax.ShapeDtypeStruct((n_pad, D_NODE), jnp.float32),
    )(hn_pad, x_pad, a.reshape(1, D_NODE), bb.reshape(1, D_NODE), alv)


def _pad_to(x, m, axis=0, value=0):
    n = x.shape[axis]
    p = (-n) % m
    if p == 0:
        return x
    widths = [(0, 0)] * x.ndim
    widths[axis] = (0, p)
    return jnp.pad(x, widths, constant_values=value)


def kernel(atom_type, pos, batch, edge_index, params):
    n = atom_type.shape[0]
    e_total = edge_index.shape[1]
    e_neigh = e_total - n
    src = edge_index[0]
    dst = edge_index[1]

    e_padf = -(-e_total // (32 * _LC)) * (32 * _LC)
    src_full = _pad_to(src, e_padf, value=0)[:e_padf]
    dst_full = _pad_to(dst, e_padf, value=0)[:e_padf]
    l2 = _sc_edge_len2(pos[:, 0], pos[:, 1], pos[:, 2], src_full, dst_full)
    el2 = l2.reshape(e_padf, 1)
    off = (jnp.arange(D_EDGE, dtype=jnp.float32) * (CUTOFF / (D_EDGE - 1)))
    off = off.reshape(1, D_EDGE)
    e_layers = [_edge_mlp_layer(el2, off, params['convs'][i])
                for i in range(N_LAYERS)]

    x = params['emb'][atom_type]
    x_pad = _pad_to(x, _NB)
    n_pad = x_pad.shape[0]

    per = 32 * _SC_C
    nchunks = -(-(-(-e_neigh // per)) // 4) * 4
    e32 = nchunks * per
    acc_rows = 10112
    dump_row = acc_rows - 8
    src_pad = _pad_to(src[:e_neigh], e32, value=0)[:e32]
    dst_pad = _pad_to(dst[:e_neigh], e32, value=dump_row)[:e32]

    for i in range(N_LAYERS):
        c = params['convs'][i]
        h0_pair = _sc_scatter(x_pad, e_layers[i], src_pad, dst_pad, n_pad,
                              acc_rows, nchunks)
        es_pad = _pad_to(e_layers[i][e_neigh:e_total], _NB)

        hn_pad, s1, s2 = _node_mlp(x_pad, h0_pair, es_pad, c,
                                   i + 1 == N_LAYERS, n)
        gn = params['gn'][i]
        mu = jnp.sum(s1, axis=(0, 1)) / n
        m2 = jnp.sum(s2, axis=(0, 1)) / n
        ms = gn['ms']
        var = m2 - 2.0 * ms * mu * mu + ms * ms * mu * mu
        inv = 1.0 / jnp.sqrt(var + 1e-5)
        a = inv * gn['g']
        bb = gn['b'] - ms * mu * inv * gn['g']
        x_pad = _norm_residual(hn_pad, x_pad, a, bb, params['alpha'][i])

    return x_pad[:n]


# speedup vs baseline: 1.1823x; 1.0233x over previous
import functools

import jax
import jax.numpy as jnp
from jax import lax
from jax.experimental import pallas as pl
from jax.experimental.pallas import tpu as pltpu
from jax.experimental.pallas import tpu_sc as plsc

N_LAYERS = 3
D_NODE = 128
D_EDGE = 64
CUTOFF = 10.0

_EB = 1024
_NB = 512


def _ln(x, g, b):
    mu = jnp.mean(x, axis=-1, keepdims=True)
    v = jnp.mean((x - mu) ** 2, axis=-1, keepdims=True)
    return (x - mu) / jnp.sqrt(v + 1e-5) * g + b


def _silu(x):
    return x * (1.0 / (1.0 + jnp.exp(-x)))


def _edge_mlp_body(el_ref, off_ref, w0_ref, b0_ref, g0_ref, bt0_ref,
                   w1_ref, b1_ref, out_ref):
    el = jnp.sqrt(el_ref[...])
    off = off_ref[...]
    step = CUTOFF / (D_EDGE - 1)
    coeff = -0.5 / (step * step)
    d = el - off
    ea = jnp.exp(coeff * d * d)
    e = jnp.dot(ea, w0_ref[...], preferred_element_type=jnp.float32) + b0_ref[...]
    onesm = jnp.full((D_NODE, D_NODE), 1.0 / D_NODE, dtype=jnp.float32)
    mu = jnp.dot(e, onesm, preferred_element_type=jnp.float32)
    xc = e - mu
    v = jnp.dot(xc * xc, onesm, preferred_element_type=jnp.float32)
    e = xc * jax.lax.rsqrt(v + 1e-5) * g0_ref[...] + bt0_ref[...]
    e = 0.5 * e * (jnp.tanh(0.5 * e) + 1.0)
    e = jnp.dot(e, w1_ref[...], preferred_element_type=jnp.float32) + b1_ref[...]
    out_ref[...] = e


def _edge_mlp_layer(el2, off, c):
    e_pad = el2.shape[0]
    nblk = e_pad // _EB
    vspec = pl.BlockSpec((1, D_NODE), lambda b: (0, 0))
    return pl.pallas_call(
        _edge_mlp_body,
        grid=(nblk,),
        in_specs=[
            pl.BlockSpec((_EB, 1), lambda b: (b, 0)),
            pl.BlockSpec((1, D_EDGE), lambda b: (0, 0)),
            pl.BlockSpec((D_EDGE, D_NODE), lambda b: (0, 0)),
            vspec, vspec, vspec,
            pl.BlockSpec((D_NODE, D_NODE), lambda b: (0, 0)),
            vspec,
        ],
        out_specs=pl.BlockSpec((_EB, D_NODE), lambda b: (b, 0)),
        out_shape=jax.ShapeDtypeStruct((e_pad, D_NODE), jnp.float32),
    )(el2, off, c['eW0'], c['eb0'].reshape(1, D_NODE),
      c['eg0'].reshape(1, D_NODE), c['ebt0'].reshape(1, D_NODE), c['eW1'],
      c['eb1'].reshape(1, D_NODE))


_SC_C = 64
_ZB = 16
_LC = 128


def _sc_edge_len2(px, py, pz, src_full, dst_full):
    e_padf = src_full.shape[0]
    nch = e_padf // (32 * _LC)

    def body(px_h, py_h, pz_h, src_h, dst_h, out_h,
             sidx, didx, sx, sy, sz, dx, dy, dz, l2b, sem):
        c = lax.axis_index("c")
        s = lax.axis_index("s")
        w = s * 2 + c

        def chunk(t, carry):
            base = (w * nch + t) * _LC
            pltpu.sync_copy(src_h.at[pl.ds(base, _LC)], sidx)
            pltpu.sync_copy(dst_h.at[pl.ds(base, _LC)], didx)
            cps = [pltpu.async_copy(px_h.at[sidx], sx, sem),
                   pltpu.async_copy(py_h.at[sidx], sy, sem),
                   pltpu.async_copy(pz_h.at[sidx], sz, sem),
                   pltpu.async_copy(px_h.at[didx], dx, sem),
                   pltpu.async_copy(py_h.at[didx], dy, sem),
                   pltpu.async_copy(pz_h.at[didx], dz, sem)]
            for cp in cps:
                cp.wait()

            def cb(j, inner):
                sl = pl.ds(j * 16, 16)
                a = sx[sl] - dx[sl]
                b = sy[sl] - dy[sl]
                g = sz[sl] - dz[sl]
                l2b[sl] = a * a + b * b + g * g
                return inner

            lax.fori_loop(0, _LC // 16, cb, 0)
            pltpu.sync_copy(l2b, out_h.at[pl.ds(base, _LC)])
            return carry

        lax.fori_loop(0, nch, chunk, 0)

    mesh = plsc.VectorSubcoreMesh(core_axis_name="c", subcore_axis_name="s")
    f = pl.kernel(
        body,
        out_type=jax.ShapeDtypeStruct((e_padf,), jnp.float32),
        mesh=mesh,
        scratch_types=[
            pltpu.VMEM((_LC,), jnp.int32),
            pltpu.VMEM((_LC,), jnp.int32),
            pltpu.VMEM((_LC,), jnp.float32),
            pltpu.VMEM((_LC,), jnp.float32),
            pltpu.VMEM((_LC,), jnp.float32),
            pltpu.VMEM((_LC,), jnp.float32),
            pltpu.VMEM((_LC,), jnp.float32),
            pltpu.VMEM((_LC,), jnp.float32),
            pltpu.VMEM((_LC,), jnp.float32),
            pltpu.SemaphoreType.DMA,
        ],
    )
    return f(px, py, pz, src_full, dst_full)


def _sc_scatter(x_pad, e_l, src_pad, dst_pad, n_rows, acc_rows, nchunks):
    nc, ns = 2, 16
    zr = acc_rows // ns

    def body(x_hbm, e_hbm, src_hbm, dst_hbm, out_hbm,
             idx0, idx1, idx2, idx3, dstv0, dstv1, dstv2, dstv3, xb0, xb1,
             eb0, eb1, mb0, mb1, acc_sh,
             semf0, semf1, semf2, semf3, semd0, semd1, semd2, semd3,
             seme0, seme1, semg0, semg1, sems0, sems1):
        idx = [idx0, idx1, idx2, idx3]
        dstv = [dstv0, dstv1, dstv2, dstv3]
        xb = [xb0, xb1]
        eb = [eb0, eb1]
        mb = [mb0, mb1]
        semf = [semf0, semf1, semf2, semf3]
        semd = [semd0, semd1, semd2, semd3]
        seme = [seme0, seme1]
        semg = [semg0, semg1]
        sems = [sems0, sems1]
        c = lax.axis_index("c")
        s = lax.axis_index("s")

        def zero_body(j, carry):
            for k in range(D_NODE // 16):
                sl = pl.ds(k * 16, 16)
                mb0[j, sl] = jnp.zeros((16,), jnp.float32)
            return carry

        lax.fori_loop(0, _SC_C, zero_body, 0)
        nz = zr // _SC_C
        for k in range(nz):
            pltpu.sync_copy(mb0, acc_sh.at[pl.ds(s * zr + k * _SC_C, _SC_C)])
        rem = zr - nz * _SC_C
        if rem:
            pltpu.sync_copy(mb0.at[pl.ds(0, rem)],
                            acc_sh.at[pl.ds(s * zr + nz * _SC_C, rem)])
        plsc.subcore_barrier()

        w = s * nc + c

        def ebase(t):
            return (w * nchunks + t) * _SC_C

        def issue_fetch_idx(t, a):
            pltpu.async_copy(src_hbm.at[pl.ds(ebase(t), _SC_C)], idx[a], semf[a])
            pltpu.async_copy(dst_hbm.at[pl.ds(ebase(t), _SC_C)], dstv[a], semd[a])

        def wait_fetch_idx(t, a):
            pltpu.make_async_copy(src_hbm.at[pl.ds(ebase(t), _SC_C)], idx[a], semf[a]).wait()
            pltpu.make_async_copy(dst_hbm.at[pl.ds(ebase(t), _SC_C)], dstv[a], semd[a]).wait()

        def issue_fetch_e(t, b):
            pltpu.async_copy(e_hbm.at[pl.ds(ebase(t), _SC_C)], eb[b], seme[b])

        def wait_fetch_e(t, b):
            pltpu.make_async_copy(e_hbm.at[pl.ds(ebase(t), _SC_C)], eb[b], seme[b]).wait()

        def issue_gather(a, b):
            pltpu.async_copy(x_hbm.at[idx[a]], xb[b], semg[b])

        def wait_gather(a, b):
            pltpu.make_async_copy(x_hbm.at[idx[a]], xb[b], semg[b]).wait()

        def issue_scatter(b, d):
            pltpu.async_copy(mb[b], acc_sh.at[dstv[d]], sems[b], add=True)

        def wait_scatter(b, d):
            pltpu.make_async_copy(mb[b], acc_sh.at[dstv[d]], sems[b]).wait()

        issue_fetch_idx(0, 0)
        issue_fetch_idx(1, 1)
        issue_fetch_e(0, 0)
        issue_fetch_e(1, 1)
        wait_fetch_idx(0, 0)
        issue_gather(0, 0)
        wait_fetch_idx(1, 1)
        issue_gather(1, 1)

        @pl.loop(0, nchunks, step=4)
        def _(t0):
            for dt in range(4):
                t = t0 + dt
                b = dt % 2
                d = dt
                dn2 = (dt + 2) % 4

                @pl.when(t >= 2)
                def _():
                    wait_scatter(b, dn2)

                @pl.when(t + 2 < nchunks)
                def _():
                    issue_fetch_idx(t + 2, dn2)

                wait_gather(d, b)
                wait_fetch_e(t, b)

                def mul_body(j, inner):
                    for k in range(D_NODE // 16):
                        sl = pl.ds(k * 16, 16)
                        mb[b][j, sl] = xb[b][j, sl] * eb[b][j, sl]
                    return inner

                lax.fori_loop(0, _SC_C, mul_body, 0)
                issue_scatter(b, d)

                @pl.when(t + 2 < nchunks)
                def _():
                    issue_fetch_e(t + 2, b)
                    wait_fetch_idx(t + 2, dn2)
                    issue_gather(dn2, b)

        wait_scatter((nchunks - 2) % 2, (nchunks - 2) % 4)
        wait_scatter((nchunks - 1) % 2, (nchunks - 1) % 4)
        plsc.subcore_barrier()
        pltpu.sync_copy(acc_sh.at[pl.ds(s * zr, zr)],
                        out_hbm.at[c, pl.ds(s * zr, zr)])

    mesh = plsc.VectorSubcoreMesh(core_axis_name="c", subcore_axis_name="s")
    f = pl.kernel(
        body,
        out_type=jax.ShapeDtypeStruct((nc, n_rows, D_NODE), jnp.float32),
        mesh=mesh,
        scratch_types=(
            [pltpu.VMEM((_SC_C,), jnp.int32) for _ in range(8)]
            + [pltpu.VMEM((_SC_C, D_NODE), jnp.float32) for _ in range(6)]
            + [pltpu.VMEM_SHARED((acc_rows, D_NODE), jnp.float32)]
            + [pltpu.SemaphoreType.DMA for _ in range(14)]
        ),
    )
    return f(x_pad, e_l, src_pad, dst_pad)


def _node_mlp_body(is_last, n_valid, x_ref, h0a_ref, h0b_ref, es_ref, ca_ref,
                   w0_ref, b0_ref, g0_ref, bt0_ref, w1_ref, b1_ref,
                   g1_ref, bt1_ref, hn_ref, s1_ref, s2_ref):
    x = x_ref[...]
    h0 = h0a_ref[0] + h0b_ref[0] + x * es_ref[...]
    h = ca_ref[...] * h0 + x
    h = jnp.dot(h, w0_ref[...], preferred_element_type=jnp.float32) + b0_ref[...]
    h = _silu(_ln(h, g0_ref[...], bt0_ref[...]))
    h = jnp.dot(h, w1_ref[...], preferred_element_type=jnp.float32) + b1_ref[...]
    if not is_last:
        h = _silu(_ln(h, g1_ref[...], bt1_ref[...]))
    hn_ref[...] = h
    row = jax.lax.broadcasted_iota(jnp.int32, h.shape, 0) + pl.program_id(0) * _NB
    hm = jnp.where(row < n_valid, h, 0.0)
    s1_ref[...] = jnp.sum(hm, axis=0, keepdims=True)[None]
    s2_ref[...] = jnp.sum(hm * hm, axis=0, keepdims=True)[None]


def _node_mlp(x_pad, h0_pair, es_pad, c, is_last, n_valid):
    n_pad = x_pad.shape[0]
    nblk = n_pad // _NB
    vspec = pl.BlockSpec((1, D_NODE), lambda b: (0, 0))
    hn, s1, s2 = pl.pallas_call(
        functools.partial(_node_mlp_body, is_last, n_valid),
        grid=(nblk,),
        in_specs=[
            pl.BlockSpec((_NB, D_NODE), lambda b: (b, 0)),
            pl.BlockSpec((1, _NB, D_NODE), lambda b: (0, b, 0)),
            pl.BlockSpec((1, _NB, D_NODE), lambda b: (1, b, 0)),
            pl.BlockSpec((_NB, D_NODE), lambda b: (b, 0)),
            vspec,
            pl.BlockSpec((D_NODE, D_NODE), lambda b: (0, 0)),
            vspec, vspec, vspec,
            pl.BlockSpec((D_NODE, D_NODE), lambda b: (0, 0)),
            vspec, vspec, vspec,
        ],
        out_specs=[
            pl.BlockSpec((_NB, D_NODE), lambda b: (b, 0)),
            pl.BlockSpec((1, 1, D_NODE), lambda b: (b, 0, 0)),
            pl.BlockSpec((1, 1, D_NODE), lambda b: (b, 0, 0)),
        ],
        out_shape=[
            jax.ShapeDtypeStruct((n_pad, D_NODE), jnp.float32),
            jax.ShapeDtypeStruct((nblk, 1, D_NODE), jnp.float32),
            jax.ShapeDtypeStruct((nblk, 1, D_NODE), jnp.float32),
        ],
    )(x_pad, h0_pair, h0_pair, es_pad,
      c['calpha'].reshape(1, D_NODE), c['nW0'], c['nb0'].reshape(1, D_NODE),
      c['ng0'].reshape(1, D_NODE), c['nbt0'].reshape(1, D_NODE), c['nW1'],
      c['nb1'].reshape(1, D_NODE), c['ng1'].reshape(1, D_NODE),
      c['nbt1'].reshape(1, D_NODE))
    return hn, s1, s2


def _norm_body(hn_ref, x_ref, a_ref, bb_ref, al_ref, out_ref):
    out_ref[...] = hn_ref[...] * a_ref[...] + x_ref[...] * al_ref[...] + bb_ref[...]


def _norm_residual(hn_pad, x_pad, a, bb, alpha):
    n_pad = x_pad.shape[0]
    nblk = n_pad // _NB
    vspec = pl.BlockSpec((1, D_NODE), lambda b: (0, 0))
    alv = jnp.broadcast_to(alpha.reshape(1, 1), (1, D_NODE))
    return pl.pallas_call(
        _norm_body,
        grid=(nblk,),
        in_specs=[
            pl.BlockSpec((_NB, D_NODE), lambda b: (b, 0)),
            pl.BlockSpec((_NB, D_NODE), lambda b: (b, 0)),
            vspec, vspec, vspec,
        ],
        out_specs=pl.BlockSpec((_NB, D_NODE), lambda b: (b, 0)),
        out_shape=jax.ShapeDtypeStruct((n_pad, D_NODE), jnp.float32),
    )(hn_pad, x_pad, a.reshape(1, D_NODE), bb.reshape(1, D_NODE), alv)


def _pad_to(x, m, axis=0, value=0):
    n = x.shape[axis]
    p = (-n) % m
    if p == 0:
        return x
    widths = [(0, 0)] * x.ndim
    widths[axis] = (0, p)
    return jnp.pad(x, widths, constant_values=value)


def kernel(atom_type, pos, batch, edge_index, params):
    n = atom_type.shape[0]
    e_total = edge_index.shape[1]
    e_neigh = e_total - n
    src = edge_index[0]
    dst = edge_index[1]

    e_padf = -(-e_total // (32 * _LC)) * (32 * _LC)
    src_full = _pad_to(src, e_padf, value=0)[:e_padf]
    dst_full = _pad_to(dst, e_padf, value=0)[:e_padf]
    l2 = _sc_edge_len2(pos[:, 0], pos[:, 1], pos[:, 2], src_full, dst_full)
    el2 = l2.reshape(e_padf, 1)
    off = (jnp.arange(D_EDGE, dtype=jnp.float32) * (CUTOFF / (D_EDGE - 1)))
    off = off.reshape(1, D_EDGE)
    e_layers = [_edge_mlp_layer(el2, off, params['convs'][i])
                for i in range(N_LAYERS)]

    x = params['emb'][atom_type]
    x_pad = _pad_to(x, _NB)
    n_pad = x_pad.shape[0]

    per = 32 * _SC_C
    nchunks = -(-(-(-e_neigh // per)) // 4) * 4
    e32 = nchunks * per
    acc_rows = 10112
    dump_row = acc_rows - 8
    src_pad = _pad_to(src[:e_neigh], e32, value=0)[:e32]
    dst_pad = _pad_to(dst[:e_neigh], e32, value=dump_row)[:e32]

    for i in range(N_LAYERS):
        c = params['convs'][i]
        h0_pair = _sc_scatter(x_pad, e_layers[i], src_pad, dst_pad, n_pad,
                              acc_rows, nchunks)
        es_pad = _pad_to(e_layers[i][e_neigh:e_total], _NB)

        hn_pad, s1, s2 = _node_mlp(x_pad, h0_pair, es_pad, c,
                                   i + 1 == N_LAYERS, n)
        gn = params['gn'][i]
        mu = jnp.sum(s1, axis=(0, 1)) / n
        m2 = jnp.sum(s2, axis=(0, 1)) / n
        ms = gn['ms']
        var = m2 - 2.0 * ms * mu * mu + ms * ms * mu * mu
        inv = 1.0 / jnp.sqrt(var + 1e-5)
        a = inv * gn['g']
        bb = gn['b'] - ms * mu * inv * gn['g']
        x_pad = _norm_residual(hn_pad, x_pad, a, bb, params['alpha'][i])

    return x_pad[:n]


# speedup vs baseline: 1.2360x; 1.0455x over previous
import functools

import jax
import jax.numpy as jnp
from jax import lax
from jax.experimental import pallas as pl
from jax.experimental.pallas import tpu as pltpu
from jax.experimental.pallas import tpu_sc as plsc

N_LAYERS = 3
D_NODE = 128
D_EDGE = 64
CUTOFF = 10.0

_EB = 1024
_NB = 512


def _ln(x, g, b):
    mu = jnp.mean(x, axis=-1, keepdims=True)
    v = jnp.mean((x - mu) ** 2, axis=-1, keepdims=True)
    return (x - mu) / jnp.sqrt(v + 1e-5) * g + b


def _silu(x):
    return x * (1.0 / (1.0 + jnp.exp(-x)))


def _edge_mlp_body(el_ref, off_ref, w0_ref, b0_ref, g0_ref, bt0_ref,
                   w1_ref, b1_ref, out_ref):
    el = jnp.sqrt(el_ref[...])
    off = off_ref[...]
    step = CUTOFF / (D_EDGE - 1)
    coeff = -0.5 / (step * step)
    d = el - off
    ea = jnp.exp(coeff * d * d)
    e = jnp.dot(ea, w0_ref[...], preferred_element_type=jnp.float32) + b0_ref[...]
    onesm = jnp.full((D_NODE, D_NODE), 1.0 / D_NODE, dtype=jnp.float32)
    mu = jnp.dot(e, onesm, preferred_element_type=jnp.float32)
    xc = e - mu
    v = jnp.dot(xc * xc, onesm, preferred_element_type=jnp.float32)
    e = xc * jax.lax.rsqrt(v + 1e-5) * g0_ref[...] + bt0_ref[...]
    e = 0.5 * e * (jnp.tanh(0.5 * e) + 1.0)
    e = jnp.dot(e, w1_ref[...], preferred_element_type=jnp.float32) + b1_ref[...]
    out_ref[...] = e


def _edge_mlp_layer(el2, off, c):
    e_pad = el2.shape[0]
    nblk = e_pad // _EB
    vspec = pl.BlockSpec((1, D_NODE), lambda b: (0, 0))
    return pl.pallas_call(
        _edge_mlp_body,
        grid=(nblk,),
        in_specs=[
            pl.BlockSpec((_EB, 1), lambda b: (b, 0)),
            pl.BlockSpec((1, D_EDGE), lambda b: (0, 0)),
            pl.BlockSpec((D_EDGE, D_NODE), lambda b: (0, 0)),
            vspec, vspec, vspec,
            pl.BlockSpec((D_NODE, D_NODE), lambda b: (0, 0)),
            vspec,
        ],
        out_specs=pl.BlockSpec((_EB, D_NODE), lambda b: (b, 0)),
        out_shape=jax.ShapeDtypeStruct((e_pad, D_NODE), jnp.float32),
    )(el2, off, c['eW0'], c['eb0'].reshape(1, D_NODE),
      c['eg0'].reshape(1, D_NODE), c['ebt0'].reshape(1, D_NODE), c['eW1'],
      c['eb1'].reshape(1, D_NODE))


_SC_C = 64
_ZB = 16
_LC = 128


def _sc_edge_len2(px, py, pz, src_full, dst_full):
    e_padf = src_full.shape[0]
    nch = e_padf // (32 * _LC)

    def body(px_h, py_h, pz_h, src_h, dst_h, out_h,
             sidx0, sidx1, didx0, didx1, g0, g1, l2b0, l2b1,
             semi0, semi1, semg0, semg1, semo0, semo1):
        sidx = [sidx0, sidx1]
        didx = [didx0, didx1]
        gb = [g0, g1]
        l2b = [l2b0, l2b1]
        semi = [semi0, semi1]
        semg = [semg0, semg1]
        semo = [semo0, semo1]
        c = lax.axis_index("c")
        s = lax.axis_index("s")
        w = s * 2 + c

        def base(t):
            return (w * nch + t) * _LC

        def issue_idx(t, b):
            pltpu.async_copy(src_h.at[pl.ds(base(t), _LC)], sidx[b], semi[b])
            pltpu.async_copy(dst_h.at[pl.ds(base(t), _LC)], didx[b], semi[b])

        def wait_idx(t, b):
            pltpu.make_async_copy(src_h.at[pl.ds(base(t), _LC)], sidx[b], semi[b]).wait()
            pltpu.make_async_copy(dst_h.at[pl.ds(base(t), _LC)], didx[b], semi[b]).wait()

        def issue_gather(b):
            pltpu.async_copy(px_h.at[sidx[b]], gb[b].at[0], semg[b])
            pltpu.async_copy(py_h.at[sidx[b]], gb[b].at[1], semg[b])
            pltpu.async_copy(pz_h.at[sidx[b]], gb[b].at[2], semg[b])
            pltpu.async_copy(px_h.at[didx[b]], gb[b].at[3], semg[b])
            pltpu.async_copy(py_h.at[didx[b]], gb[b].at[4], semg[b])
            pltpu.async_copy(pz_h.at[didx[b]], gb[b].at[5], semg[b])

        def wait_gather(b):
            pltpu.make_async_copy(px_h.at[sidx[b]], gb[b].at[0], semg[b]).wait()
            pltpu.make_async_copy(py_h.at[sidx[b]], gb[b].at[1], semg[b]).wait()
            pltpu.make_async_copy(pz_h.at[sidx[b]], gb[b].at[2], semg[b]).wait()
            pltpu.make_async_copy(px_h.at[didx[b]], gb[b].at[3], semg[b]).wait()
            pltpu.make_async_copy(py_h.at[didx[b]], gb[b].at[4], semg[b]).wait()
            pltpu.make_async_copy(pz_h.at[didx[b]], gb[b].at[5], semg[b]).wait()

        def issue_out(t, b):
            pltpu.async_copy(l2b[b], out_h.at[pl.ds(base(t), _LC)], semo[b])

        def wait_out(t, b):
            pltpu.make_async_copy(l2b[b], out_h.at[pl.ds(base(t), _LC)], semo[b]).wait()

        issue_idx(0, 0)
        issue_idx(1, 1)
        wait_idx(0, 0)
        issue_gather(0)
        wait_idx(1, 1)
        issue_gather(1)

        @pl.loop(0, nch, step=2)
        def _(t0):
            for dt in range(2):
                t = t0 + dt
                b = dt

                @pl.when(t >= 2)
                def _():
                    wait_out(t - 2, b)

                wait_gather(b)

                def cb(j, inner):
                    sl = pl.ds(j * 16, 16)
                    a = gb[b][0, sl] - gb[b][3, sl]
                    e = gb[b][1, sl] - gb[b][4, sl]
                    g = gb[b][2, sl] - gb[b][5, sl]
                    l2b[b][sl] = a * a + e * e + g * g
                    return inner

                lax.fori_loop(0, _LC // 16, cb, 0)
                issue_out(t, b)

                @pl.when(t + 2 < nch)
                def _():
                    issue_idx(t + 2, b)
                    wait_idx(t + 2, b)
                    issue_gather(b)

        wait_out(nch - 2, 0)
        wait_out(nch - 1, 1)

    mesh = plsc.VectorSubcoreMesh(core_axis_name="c", subcore_axis_name="s")
    f = pl.kernel(
        body,
        out_type=jax.ShapeDtypeStruct((e_padf,), jnp.float32),
        mesh=mesh,
        scratch_types=(
            [pltpu.VMEM((_LC,), jnp.int32) for _ in range(4)]
            + [pltpu.VMEM((6, _LC), jnp.float32) for _ in range(2)]
            + [pltpu.VMEM((_LC,), jnp.float32) for _ in range(2)]
            + [pltpu.SemaphoreType.DMA for _ in range(6)]
        ),
    )
    return f(px, py, pz, src_full, dst_full)


def _sc_scatter(x_pad, e_l, src_pad, dst_pad, n_rows, acc_rows, nchunks):
    nc, ns = 2, 16
    zr = acc_rows // ns

    def body(x_hbm, e_hbm, src_hbm, dst_hbm, out_hbm,
             idx0, idx1, idx2, idx3, dstv0, dstv1, dstv2, dstv3, xb0, xb1,
             eb0, eb1, mb0, mb1, acc_sh,
             semf0, semf1, semf2, semf3, semd0, semd1, semd2, semd3,
             seme0, seme1, semg0, semg1, sems0, sems1):
        idx = [idx0, idx1, idx2, idx3]
        dstv = [dstv0, dstv1, dstv2, dstv3]
        xb = [xb0, xb1]
        eb = [eb0, eb1]
        mb = [mb0, mb1]
        semf = [semf0, semf1, semf2, semf3]
        semd = [semd0, semd1, semd2, semd3]
        seme = [seme0, seme1]
        semg = [semg0, semg1]
        sems = [sems0, sems1]
        c = lax.axis_index("c")
        s = lax.axis_index("s")

        def zero_body(j, carry):
            for k in range(D_NODE // 16):
                sl = pl.ds(k * 16, 16)
                mb0[j, sl] = jnp.zeros((16,), jnp.float32)
            return carry

        lax.fori_loop(0, _SC_C, zero_body, 0)
        nz = zr // _SC_C
        for k in range(nz):
            pltpu.sync_copy(mb0, acc_sh.at[pl.ds(s * zr + k * _SC_C, _SC_C)])
        rem = zr - nz * _SC_C
        if rem:
            pltpu.sync_copy(mb0.at[pl.ds(0, rem)],
                            acc_sh.at[pl.ds(s * zr + nz * _SC_C, rem)])
        plsc.subcore_barrier()

        w = s * nc + c

        def ebase(t):
            return (w * nchunks + t) * _SC_C

        def issue_fetch_idx(t, a):
            pltpu.async_copy(src_hbm.at[pl.ds(ebase(t), _SC_C)], idx[a], semf[a])
            pltpu.async_copy(dst_hbm.at[pl.ds(ebase(t), _SC_C)], dstv[a], semd[a])

        def wait_fetch_idx(t, a):
            pltpu.make_async_copy(src_hbm.at[pl.ds(ebase(t), _SC_C)], idx[a], semf[a]).wait()
            pltpu.make_async_copy(dst_hbm.at[pl.ds(ebase(t), _SC_C)], dstv[a], semd[a]).wait()

        def issue_fetch_e(t, b):
            pltpu.async_copy(e_hbm.at[pl.ds(ebase(t), _SC_C)], eb[b], seme[b])

        def wait_fetch_e(t, b):
            pltpu.make_async_copy(e_hbm.at[pl.ds(ebase(t), _SC_C)], eb[b], seme[b]).wait()

        def issue_gather(a, b):
            pltpu.async_copy(x_hbm.at[idx[a]], xb[b], semg[b])

        def wait_gather(a, b):
            pltpu.make_async_copy(x_hbm.at[idx[a]], xb[b], semg[b]).wait()

        def issue_scatter(b, d):
            pltpu.async_copy(mb[b], acc_sh.at[dstv[d]], sems[b], add=True)

        def wait_scatter(b, d):
            pltpu.make_async_copy(mb[b], acc_sh.at[dstv[d]], sems[b]).wait()

        issue_fetch_idx(0, 0)
        issue_fetch_idx(1, 1)
        issue_fetch_e(0, 0)
        issue_fetch_e(1, 1)
        wait_fetch_idx(0, 0)
        issue_gather(0, 0)
        wait_fetch_idx(1, 1)
        issue_gather(1, 1)

        @pl.loop(0, nchunks, step=4)
        def _(t0):
            for dt in range(4):
                t = t0 + dt
                b = dt % 2
                d = dt
                dn2 = (dt + 2) % 4

                @pl.when(t >= 2)
                def _():
                    wait_scatter(b, dn2)

                @pl.when(t + 2 < nchunks)
                def _():
                    issue_fetch_idx(t + 2, dn2)

                wait_gather(d, b)
                wait_fetch_e(t, b)

                def mul_body(j, inner):
                    for k in range(D_NODE // 16):
                        sl = pl.ds(k * 16, 16)
                        mb[b][j, sl] = xb[b][j, sl] * eb[b][j, sl]
                    return inner

                lax.fori_loop(0, _SC_C, mul_body, 0)
                issue_scatter(b, d)

                @pl.when(t + 2 < nchunks)
                def _():
                    issue_fetch_e(t + 2, b)
                    wait_fetch_idx(t + 2, dn2)
                    issue_gather(dn2, b)

        wait_scatter((nchunks - 2) % 2, (nchunks - 2) % 4)
        wait_scatter((nchunks - 1) % 2, (nchunks - 1) % 4)
        plsc.subcore_barrier()
        pltpu.sync_copy(acc_sh.at[pl.ds(s * zr, zr)],
                        out_hbm.at[c, pl.ds(s * zr, zr)])

    mesh = plsc.VectorSubcoreMesh(core_axis_name="c", subcore_axis_name="s")
    f = pl.kernel(
        body,
        out_type=jax.ShapeDtypeStruct((nc, n_rows, D_NODE), jnp.float32),
        mesh=mesh,
        scratch_types=(
            [pltpu.VMEM((_SC_C,), jnp.int32) for _ in range(8)]
            + [pltpu.VMEM((_SC_C, D_NODE), jnp.float32) for _ in range(6)]
            + [pltpu.VMEM_SHARED((acc_rows, D_NODE), jnp.float32)]
            + [pltpu.SemaphoreType.DMA for _ in range(14)]
        ),
    )
    return f(x_pad, e_l, src_pad, dst_pad)


def _node_mlp_body(is_last, n_valid, x_ref, h0a_ref, h0b_ref, es_ref, ca_ref,
                   w0_ref, b0_ref, g0_ref, bt0_ref, w1_ref, b1_ref,
                   g1_ref, bt1_ref, hn_ref, s1_ref, s2_ref):
    x = x_ref[...]
    h0 = h0a_ref[0] + h0b_ref[0] + x * es_ref[...]
    h = ca_ref[...] * h0 + x
    h = jnp.dot(h, w0_ref[...], preferred_element_type=jnp.float32) + b0_ref[...]
    h = _silu(_ln(h, g0_ref[...], bt0_ref[...]))
    h = jnp.dot(h, w1_ref[...], preferred_element_type=jnp.float32) + b1_ref[...]
    if not is_last:
        h = _silu(_ln(h, g1_ref[...], bt1_ref[...]))
    hn_ref[...] = h
    row = jax.lax.broadcasted_iota(jnp.int32, h.shape, 0) + pl.program_id(0) * _NB
    hm = jnp.where(row < n_valid, h, 0.0)
    s1_ref[...] = jnp.sum(hm, axis=0, keepdims=True)[None]
    s2_ref[...] = jnp.sum(hm * hm, axis=0, keepdims=True)[None]


def _node_mlp(x_pad, h0_pair, es_pad, c, is_last, n_valid):
    n_pad = x_pad.shape[0]
    nblk = n_pad // _NB
    vspec = pl.BlockSpec((1, D_NODE), lambda b: (0, 0))
    hn, s1, s2 = pl.pallas_call(
        functools.partial(_node_mlp_body, is_last, n_valid),
        grid=(nblk,),
        in_specs=[
            pl.BlockSpec((_NB, D_NODE), lambda b: (b, 0)),
            pl.BlockSpec((1, _NB, D_NODE), lambda b: (0, b, 0)),
            pl.BlockSpec((1, _NB, D_NODE), lambda b: (1, b, 0)),
            pl.BlockSpec((_NB, D_NODE), lambda b: (b, 0)),
            vspec,
            pl.BlockSpec((D_NODE, D_NODE), lambda b: (0, 0)),
            vspec, vspec, vspec,
            pl.BlockSpec((D_NODE, D_NODE), lambda b: (0, 0)),
            vspec, vspec, vspec,
        ],
        out_specs=[
            pl.BlockSpec((_NB, D_NODE), lambda b: (b, 0)),
            pl.BlockSpec((1, 1, D_NODE), lambda b: (b, 0, 0)),
            pl.BlockSpec((1, 1, D_NODE), lambda b: (b, 0, 0)),
        ],
        out_shape=[
            jax.ShapeDtypeStruct((n_pad, D_NODE), jnp.float32),
            jax.ShapeDtypeStruct((nblk, 1, D_NODE), jnp.float32),
            jax.ShapeDtypeStruct((nblk, 1, D_NODE), jnp.float32),
        ],
    )(x_pad, h0_pair, h0_pair, es_pad,
      c['calpha'].reshape(1, D_NODE), c['nW0'], c['nb0'].reshape(1, D_NODE),
      c['ng0'].reshape(1, D_NODE), c['nbt0'].reshape(1, D_NODE), c['nW1'],
      c['nb1'].reshape(1, D_NODE), c['ng1'].reshape(1, D_NODE),
      c['nbt1'].reshape(1, D_NODE))
    return hn, s1, s2


def _norm_body(hn_ref, x_ref, a_ref, bb_ref, al_ref, out_ref):
    out_ref[...] = hn_ref[...] * a_ref[...] + x_ref[...] * al_ref[...] + bb_ref[...]


def _norm_residual(hn_pad, x_pad, a, bb, alpha):
    n_pad = x_pad.shape[0]
    nblk = n_pad // _NB
    vspec = pl.BlockSpec((1, D_NODE), lambda b: (0, 0))
    alv = jnp.broadcast_to(alpha.reshape(1, 1), (1, D_NODE))
    return pl.pallas_call(
        _norm_body,
        grid=(nblk,),
        in_specs=[
            pl.BlockSpec((_NB, D_NODE), lambda b: (b, 0)),
            pl.BlockSpec((_NB, D_NODE), lambda b: (b, 0)),
            vspec, vspec, vspec,
        ],
        out_specs=pl.BlockSpec((_NB, D_NODE), lambda b: (b, 0)),
        out_shape=jax.ShapeDtypeStruct((n_pad, D_NODE), jnp.float32),
    )(hn_pad, x_pad, a.reshape(1, D_NODE), bb.reshape(1, D_NODE), alv)


def _pad_to(x, m, axis=0, value=0):
    n = x.shape[axis]
    p = (-n) % m
    if p == 0:
        return x
    widths = [(0, 0)] * x.ndim
    widths[axis] = (0, p)
    return jnp.pad(x, widths, constant_values=value)


def kernel(atom_type, pos, batch, edge_index, params):
    n = atom_type.shape[0]
    e_total = edge_index.shape[1]
    e_neigh = e_total - n
    src = edge_index[0]
    dst = edge_index[1]

    e_padf = -(-e_total // (64 * _LC)) * (64 * _LC)
    src_full = _pad_to(src, e_padf, value=0)[:e_padf]
    dst_full = _pad_to(dst, e_padf, value=0)[:e_padf]
    l2 = _sc_edge_len2(pos[:, 0], pos[:, 1], pos[:, 2], src_full, dst_full)
    el2 = l2.reshape(e_padf, 1)
    off = (jnp.arange(D_EDGE, dtype=jnp.float32) * (CUTOFF / (D_EDGE - 1)))
    off = off.reshape(1, D_EDGE)
    e_layers = [_edge_mlp_layer(el2, off, params['convs'][i])
                for i in range(N_LAYERS)]

    x = params['emb'][atom_type]
    x_pad = _pad_to(x, _NB)
    n_pad = x_pad.shape[0]

    per = 32 * _SC_C
    nchunks = -(-(-(-e_neigh // per)) // 4) * 4
    e32 = nchunks * per
    acc_rows = 10112
    dump_row = acc_rows - 8
    src_pad = _pad_to(src[:e_neigh], e32, value=0)[:e32]
    dst_pad = _pad_to(dst[:e_neigh], e32, value=dump_row)[:e32]

    for i in range(N_LAYERS):
        c = params['convs'][i]
        h0_pair = _sc_scatter(x_pad, e_layers[i], src_pad, dst_pad, n_pad,
                              acc_rows, nchunks)
        es_pad = _pad_to(e_layers[i][e_neigh:e_total], _NB)

        hn_pad, s1, s2 = _node_mlp(x_pad, h0_pair, es_pad, c,
                                   i + 1 == N_LAYERS, n)
        gn = params['gn'][i]
        mu = jnp.sum(s1, axis=(0, 1)) / n
        m2 = jnp.sum(s2, axis=(0, 1)) / n
        ms = gn['ms']
        var = m2 - 2.0 * ms * mu * mu + ms * ms * mu * mu
        inv = 1.0 / jnp.sqrt(var + 1e-5)
        a = inv * gn['g']
        bb = gn['b'] - ms * mu * inv * gn['g']
        x_pad = _norm_residual(hn_pad, x_pad, a, bb, params['alpha'][i])

    return x_pad[:n]
